# baseline (speedup 1.0000x reference)
import dataclasses
import functools

import jax
import jax.numpy as jnp
from jax import lax
from jax.experimental import pallas as pl
from jax.experimental.pallas import tpu as pltpu
from jax.experimental.pallas import tpu_sc as plsc

L = 16
SB = 8
BB = 1024


def _lookup_sc(idx_t, table_rep, F):
    S, B = idx_t.shape
    mesh = plsc.VectorSubcoreMesh(core_axis_name="c", subcore_axis_name="s")

    cp = pltpu.CompilerParams()
    if "needs_layout_passes" in pltpu.CompilerParams.__dataclass_fields__:
        cp = dataclasses.replace(cp, needs_layout_passes=False)

    @functools.partial(
        pl.kernel,
        out_type=jax.ShapeDtypeStruct((F, S, B), jnp.float32),
        mesh=mesh,
        scratch_types=[pltpu.VMEM(table_rep.shape, jnp.float32)],
        compiler_params=cp,
    )
    def run(table_hbm, idx_hbm, out_hbm, table_v):
        pltpu.sync_copy(table_hbm, table_v)

        def body(idx_v, out_v):
            lane = lax.iota(jnp.int32, L)

            @pl.loop(0, SB)
            def _(s):
                @plsc.parallel_loop(0, BB, step=L, unroll=8)
                def _(b):
                    iv = idx_v[s, pl.ds(b, L)]
                    base = iv * (F * L) + lane
                    for f in range(F):
                        vals = plsc.load_gather(table_v, [base + f * L])
                        out_v[f, s, pl.ds(b, L)] = vals

        pltpu.emit_pipeline(
            body,
            grid=(S // SB, B // BB),
            in_specs=[pl.BlockSpec((SB, BB), lambda i, j: (i, j))],
            out_specs=[pl.BlockSpec((F, SB, BB), lambda i, j: (0, i, j))],
            core_axis_name=("c", "s"),
            dimension_semantics=(pltpu.PARALLEL, pltpu.PARALLEL),
        )(idx_hbm, out_hbm)

    return run(table_rep, idx_t)


def kernel(inputs, seq_vectors):
    B, S = inputs.shape
    V, F = seq_vectors.shape
    idx_t = inputs.T
    table_rep = jnp.repeat(seq_vectors.reshape(-1), L)
    out_t = _lookup_sc(idx_t, table_rep, F)
    return out_t.transpose(2, 1, 0)


# speedup vs baseline: 1.0199x; 1.0199x over previous
import dataclasses
import functools

import jax
import jax.numpy as jnp
from jax import lax
from jax.experimental import pallas as pl
from jax.experimental.pallas import tpu as pltpu
from jax.experimental.pallas import tpu_sc as plsc

L = 16
SB = 8
BB = 512


def _lookup_sc(idx_t, table_rep, F):
    S, B = idx_t.shape
    mesh = plsc.VectorSubcoreMesh(core_axis_name="c", subcore_axis_name="s")

    cp = pltpu.CompilerParams()
    if "needs_layout_passes" in pltpu.CompilerParams.__dataclass_fields__:
        cp = dataclasses.replace(cp, needs_layout_passes=False)

    @functools.partial(
        pl.kernel,
        out_type=jax.ShapeDtypeStruct((F, S, B), jnp.float32),
        mesh=mesh,
        scratch_types=[pltpu.VMEM(table_rep.shape, jnp.float32)],
        compiler_params=cp,
    )
    def run(table_hbm, idx_hbm, out_hbm, table_v):
        pltpu.sync_copy(table_hbm, table_v)

        def body(idx_v, out_v):
            lane = lax.iota(jnp.int32, L)

            @plsc.parallel_loop(0, SB * BB, step=L, unroll=8)
            def _(g):
                s = g // BB
                b = g % BB
                iv = idx_v[s, pl.ds(b, L)]
                base = iv * (F * L) + lane
                for f in range(F):
                    vals = plsc.load_gather(table_v, [base + f * L])
                    out_v[f, s, pl.ds(b, L)] = vals

        pltpu.emit_pipeline(
            body,
            grid=(S // SB, B // BB),
            in_specs=[pl.BlockSpec((SB, BB), lambda i, j: (i, j))],
            out_specs=[pl.BlockSpec((F, SB, BB), lambda i, j: (0, i, j))],
            core_axis_name=("c", "s"),
            dimension_semantics=(pltpu.PARALLEL, pltpu.PARALLEL),
        )(idx_hbm, out_hbm)

    return run(table_rep, idx_t)


def kernel(inputs, seq_vectors):
    B, S = inputs.shape
    V, F = seq_vectors.shape
    idx_t = inputs.T
    table_rep = jnp.repeat(seq_vectors.reshape(-1), L)
    out_t = _lookup_sc(idx_t, table_rep, F)
    return out_t.transpose(2, 1, 0)


# speedup vs baseline: 1.0440x; 1.0236x over previous
import dataclasses
import functools

import jax
import jax.numpy as jnp
from jax import lax
from jax.experimental import pallas as pl
from jax.experimental.pallas import tpu as pltpu
from jax.experimental.pallas import tpu_sc as plsc

L = 16
SB = 40
BB = 256


def _lookup_sc(idx_t, table_rep, F):
    S, B = idx_t.shape
    mesh = plsc.VectorSubcoreMesh(core_axis_name="c", subcore_axis_name="s")

    cp = pltpu.CompilerParams()
    if "needs_layout_passes" in pltpu.CompilerParams.__dataclass_fields__:
        cp = dataclasses.replace(cp, needs_layout_passes=False)

    @functools.partial(
        pl.kernel,
        out_type=jax.ShapeDtypeStruct((F, S, B), jnp.float32),
        mesh=mesh,
        scratch_types=[pltpu.VMEM(table_rep.shape, jnp.float32)],
        compiler_params=cp,
    )
    def run(table_hbm, idx_hbm, out_hbm, table_v):
        pltpu.sync_copy(table_hbm, table_v)

        def body(idx_v, out_v):
            lane = lax.iota(jnp.int32, L)

            @plsc.parallel_loop(0, SB * BB, step=L, unroll=8)
            def _(g):
                s = g // BB
                b = g % BB
                iv = idx_v[s, pl.ds(b, L)]
                base = iv * (F * L) + lane
                for f in range(F):
                    vals = plsc.load_gather(table_v, [base + f * L])
                    out_v[f, s, pl.ds(b, L)] = vals

        pltpu.emit_pipeline(
            body,
            grid=(S // SB, B // BB),
            in_specs=[pl.BlockSpec((SB, BB), lambda i, j: (i, j))],
            out_specs=[pl.BlockSpec((F, SB, BB), lambda i, j: (0, i, j))],
            core_axis_name=("c", "s"),
            dimension_semantics=(pltpu.PARALLEL, pltpu.PARALLEL),
        )(idx_hbm, out_hbm)

    return run(table_rep, idx_t)


def kernel(inputs, seq_vectors):
    B, S = inputs.shape
    V, F = seq_vectors.shape
    idx_t = inputs.T
    table_rep = jnp.repeat(seq_vectors.reshape(-1), L)
    out_t = _lookup_sc(idx_t, table_rep, F)
    return out_t.transpose(2, 1, 0)
